# baseline (speedup 1.0000x reference)
import functools

import jax
import jax.numpy as jnp
from jax import lax
from jax.experimental import pallas as pl
from jax.experimental.pallas import tpu as pltpu
from jax.experimental.pallas import tpu_sc as plsc

B = 16384
ENUM = 32
IN_FEATS = 80

STAND_OFF = 0
TAXI_OFF = 72
HR_OFF = 1080
WEEK_OFF = 1128
PS_ROWS = 1136

PC_CHUNK = 12504
PC_GRID = 8
PC_ROWS = PC_CHUNK * PC_GRID
PC_COLS = 8

NC, NS, LANES = 2, 16, 16
NW = NC * NS
ROWS_PER_W = B // NW
GROUPS = ROWS_PER_W // LANES
IDX_CHUNK = 128
N_CHUNKS = ROWS_PER_W // IDX_CHUNK


def _tc_body(x_ref, wi_ref, b_ref, st_ref, ws_ref, tx_ref, wt_ref,
             hr_ref, wh_ref, wk_ref, ww_ref, ce_ref, wc_ref,
             dense_ref, ps_ref, pc_ref):
    @pl.when(pl.program_id(0) == 0)
    def _():
        dense_ref[...] = (
            jnp.dot(x_ref[...], wi_ref[...], preferred_element_type=jnp.float32)
            + b_ref[...]
        )
        ps_ref[0:72, :] = jnp.dot(st_ref[...], ws_ref[...],
                                  preferred_element_type=jnp.float32)
        ps_ref[72:1080, :] = jnp.dot(tx_ref[...], wt_ref[...],
                                     preferred_element_type=jnp.float32)
        ps_ref[1080:1128, :] = jnp.dot(hr_ref[...], wh_ref[...],
                                       preferred_element_type=jnp.float32)
        ps_ref[1128:1136, :] = jnp.dot(wk_ref[...], ww_ref[...],
                                       preferred_element_type=jnp.float32)

    pc_ref[...] = jnp.dot(ce_ref[...], wc_ref[...],
                          preferred_element_type=jnp.float32)


def _sc_body(dense_hbm, callidx_hbm, stand_hbm, taxi_hbm, hr_hbm, week_hbm,
             ps_hbm, pc_hbm, out_hbm,
             call_v, stand_v, taxi_v, hr_v, week_v,
             dense_v, ps_v, crow_v, out_v, sem):
    wid = lax.axis_index("s") * NC + lax.axis_index("c")
    base = wid * ROWS_PER_W

    pltpu.sync_copy(callidx_hbm.at[wid], call_v)
    pltpu.sync_copy(stand_hbm.at[pl.ds(base, ROWS_PER_W)], stand_v)
    pltpu.sync_copy(taxi_hbm.at[pl.ds(base, ROWS_PER_W)], taxi_v)
    pltpu.sync_copy(hr_hbm.at[pl.ds(base, ROWS_PER_W)], hr_v)
    pltpu.sync_copy(week_hbm.at[pl.ds(base, ROWS_PER_W)], week_v)
    pltpu.sync_copy(dense_hbm.at[pl.ds(base * 2, 2 * ROWS_PER_W)], dense_v)
    pltpu.sync_copy(ps_hbm, ps_v)

    copies = [
        pltpu.async_copy(
            pc_hbm.at[call_v.at[j]],
            crow_v.at[j],
            sem,
        )
        for j in range(N_CHUNKS)
    ]
    for c in copies:
        c.wait()

    lanes = lax.iota(jnp.int32, LANES)
    lanes2 = lanes * 2
    c0 = jnp.zeros((LANES,), jnp.int32)
    c1 = jnp.ones((LANES,), jnp.int32)

    def group(g, carry):
        out_base = lanes2 + g * (2 * LANES)
        acc0 = plsc.load_gather(dense_v, [out_base])
        acc1 = plsc.load_gather(dense_v, [out_base + 1])
        jv = jnp.full((LANES,), g // 8, jnp.int32)
        rv = lanes + (g % 8) * LANES
        acc0 = acc0 + plsc.load_gather(crow_v, [jv, rv, c0])
        acc1 = acc1 + plsc.load_gather(crow_v, [jv, rv, c1])
        for idx_ref in (stand_v, taxi_v, hr_v, week_v):
            iv = idx_ref[pl.ds(g * LANES, LANES)]
            acc0 = acc0 + plsc.load_gather(ps_v, [iv])
            acc1 = acc1 + plsc.load_gather(ps_v, [iv + 1])
        plsc.store_scatter(out_v, [out_base], acc0)
        plsc.store_scatter(out_v, [out_base + 1], acc1)
        return carry

    lax.fori_loop(0, GROUPS, group, 0)
    pltpu.sync_copy(out_v, out_hbm.at[pl.ds(base * 2, 2 * ROWS_PER_W)])


_sc_call = functools.partial(
    pl.kernel,
    out_type=jax.ShapeDtypeStruct((B * 2,), jnp.float32),
    compiler_params=pltpu.CompilerParams(
        needs_layout_passes=False, use_tc_tiling_on_sc=False,
    ),
    mesh=plsc.VectorSubcoreMesh(
        core_axis_name="c", subcore_axis_name="s",
        num_cores=NC, num_subcores=NS,
    ),
    scratch_types=[
        pltpu.VMEM((N_CHUNKS, IDX_CHUNK), jnp.int32),
        pltpu.VMEM((ROWS_PER_W,), jnp.int32),
        pltpu.VMEM((ROWS_PER_W,), jnp.int32),
        pltpu.VMEM((ROWS_PER_W,), jnp.int32),
        pltpu.VMEM((ROWS_PER_W,), jnp.int32),
        pltpu.VMEM((2 * ROWS_PER_W,), jnp.float32),
        pltpu.VMEM((2 * PS_ROWS,), jnp.float32),
        pltpu.VMEM((N_CHUNKS, IDX_CHUNK, PC_COLS), jnp.float32),
        pltpu.VMEM((2 * ROWS_PER_W,), jnp.float32),
        pltpu.SemaphoreType.DMA,
    ],
)(_sc_body)


def kernel(inputs, call, stand, taxi, hr, week, callemb, standemb, taxiemb,
           hremb, weekemb, W, b):
    f32 = jnp.float32
    i32 = jnp.int32
    wi = W[:, :IN_FEATS].T
    wc = W[:, IN_FEATS:IN_FEATS + ENUM].T
    wc8 = jnp.pad(wc, ((0, 0), (0, PC_COLS - 2)))
    ws = W[:, 112:144].T
    wt = W[:, 144:176].T
    wh = W[:, 176:208].T
    ww = W[:, 208:240].T

    st_p = jnp.pad(standemb, ((0, 72 - 65), (0, 0)))
    tx_p = jnp.pad(taxiemb, ((0, 1008 - 1000), (0, 0)))
    wk_p = jnp.pad(weekemb, ((0, 8 - 7), (0, 0)))

    dense, ps, pc = pl.pallas_call(
        _tc_body,
        grid=(PC_GRID,),
        in_specs=[
            pl.BlockSpec((B, IN_FEATS), lambda i: (0, 0)),
            pl.BlockSpec((IN_FEATS, 2), lambda i: (0, 0)),
            pl.BlockSpec((1, 2), lambda i: (0, 0)),
            pl.BlockSpec((72, ENUM), lambda i: (0, 0)),
            pl.BlockSpec((ENUM, 2), lambda i: (0, 0)),
            pl.BlockSpec((1008, ENUM), lambda i: (0, 0)),
            pl.BlockSpec((ENUM, 2), lambda i: (0, 0)),
            pl.BlockSpec((48, ENUM), lambda i: (0, 0)),
            pl.BlockSpec((ENUM, 2), lambda i: (0, 0)),
            pl.BlockSpec((8, ENUM), lambda i: (0, 0)),
            pl.BlockSpec((ENUM, 2), lambda i: (0, 0)),
            pl.BlockSpec((PC_CHUNK, ENUM), lambda i: (i, 0)),
            pl.BlockSpec((ENUM, PC_COLS), lambda i: (0, 0)),
        ],
        out_specs=[
            pl.BlockSpec((B, 2), lambda i: (0, 0)),
            pl.BlockSpec((PS_ROWS, 2), lambda i: (0, 0)),
            pl.BlockSpec((PC_CHUNK, PC_COLS), lambda i: (i, 0)),
        ],
        out_shape=[
            jax.ShapeDtypeStruct((B, 2), f32),
            jax.ShapeDtypeStruct((PS_ROWS, 2), f32),
            jax.ShapeDtypeStruct((PC_ROWS, PC_COLS), f32),
        ],
    )(inputs, wi, b.reshape(1, 2), st_p, ws, tx_p, wt, hremb, wh, wk_p, ww,
      callemb, wc8)

    callidx = (call.astype(i32) + 1).reshape(NW, N_CHUNKS, IDX_CHUNK)
    standg = (stand.astype(i32) + (1 + STAND_OFF)) * 2
    taxig = (taxi.astype(i32) + TAXI_OFF) * 2
    hrg = (hr.astype(i32) + HR_OFF) * 2
    weekg = (week.astype(i32) + WEEK_OFF) * 2

    out_flat = _sc_call(dense.reshape(-1), callidx, standg, taxig, hrg, weekg,
                        ps.reshape(-1), pc)
    return out_flat.reshape(B, 2)


# speedup vs baseline: 1.6192x; 1.6192x over previous
import functools

import jax
import jax.numpy as jnp
from jax import lax
from jax.experimental import pallas as pl
from jax.experimental.pallas import tpu as pltpu
from jax.experimental.pallas import tpu_sc as plsc

B = 16384
ENUM = 32
IN_FEATS = 80

STAND_ROWS = 65
TAXI_OFF = 72
HR_OFF = 1080
WEEK_OFF = 1128
PS_ROWS = 1136

B_CHUNK = 2048
PC_CHUNK = 13312
PC_GRID = 8
PC_ROWS = PC_CHUNK * PC_GRID

NC, NS, LANES = 2, 16, 16
NW = NC * NS
ROWS_PER_W = B // NW
GROUPS = ROWS_PER_W // LANES
IDX_CHUNK = 128
N_CHUNKS = ROWS_PER_W // IDX_CHUNK

_DOT11 = (((1,), (1,)), ((), ()))


def _tc_body(x_ref, w_ref, b_ref, sm_ref, ce_ref,
             d0_ref, d1_ref, ps0_ref, ps1_ref, pc0_ref, pc1_ref):
    f32 = jnp.float32
    w = w_ref[...]

    d = lax.dot_general(w[:, 0:IN_FEATS], x_ref[...], _DOT11,
                        preferred_element_type=f32)
    d0_ref[...] = d[0] + b_ref[0, 0]
    d1_ref[...] = d[1] + b_ref[0, 1]

    pc = lax.dot_general(w[:, IN_FEATS:IN_FEATS + ENUM], ce_ref[...], _DOT11,
                         preferred_element_type=f32)
    pc0_ref[...] = pc[0]
    pc1_ref[...] = pc[1]

    @pl.when(pl.program_id(0) == 0)
    def _():
        for r0, rows, woff in ((0, 72, 112), (TAXI_OFF, 1008, 144),
                               (HR_OFF, 48, 176), (WEEK_OFF, 8, 208)):
            p = lax.dot_general(w[:, woff:woff + ENUM], sm_ref[r0:r0 + rows, :],
                                _DOT11, preferred_element_type=f32)
            ps0_ref[r0:r0 + rows] = p[0]
            ps1_ref[r0:r0 + rows] = p[1]


def _sc_body(d0_hbm, d1_hbm, call_hbm, stand_hbm, taxi_hbm, hr_hbm, week_hbm,
             ps0_hbm, ps1_hbm, pc0_hbm, pc1_hbm, out_hbm,
             call_v, st_v, tx_v, hr_v, wk_v,
             d0_v, d1_v, ps0_v, ps1_v, c0_v, c1_v, out_v, sem):
    wid = lax.axis_index("s") * NC + lax.axis_index("c")
    base = wid * ROWS_PER_W

    pltpu.sync_copy(call_hbm.at[pl.ds(base, ROWS_PER_W)], call_v)
    pltpu.sync_copy(stand_hbm.at[pl.ds(base, ROWS_PER_W)], st_v)
    pltpu.sync_copy(taxi_hbm.at[pl.ds(base, ROWS_PER_W)], tx_v)
    pltpu.sync_copy(hr_hbm.at[pl.ds(base, ROWS_PER_W)], hr_v)
    pltpu.sync_copy(week_hbm.at[pl.ds(base, ROWS_PER_W)], wk_v)
    pltpu.sync_copy(d0_hbm.at[pl.ds(base, ROWS_PER_W)], d0_v)
    pltpu.sync_copy(d1_hbm.at[pl.ds(base, ROWS_PER_W)], d1_v)
    pltpu.sync_copy(ps0_hbm, ps0_v)
    pltpu.sync_copy(ps1_hbm, ps1_v)

    for k in range(GROUPS):
        sl = pl.ds(k * LANES, LANES)
        call_v[sl] = call_v[sl] + 1

    copies = []
    for j in range(N_CHUNKS):
        sl = pl.ds(j * IDX_CHUNK, IDX_CHUNK)
        copies.append(pltpu.async_copy(pc0_hbm.at[call_v.at[sl]], c0_v.at[sl], sem))
        copies.append(pltpu.async_copy(pc1_hbm.at[call_v.at[sl]], c1_v.at[sl], sem))
    for c in copies:
        c.wait()

    lanes = lax.iota(jnp.int32, LANES)
    col0 = jnp.zeros((LANES,), jnp.int32)
    col1 = jnp.ones((LANES,), jnp.int32)

    def group(g, carry):
        sl = pl.ds(g * LANES, LANES)
        acc0 = d0_v[sl] + c0_v[sl]
        acc1 = d1_v[sl] + c1_v[sl]
        for idx_ref, off in ((st_v, 1), (tx_v, TAXI_OFF), (hr_v, HR_OFF),
                             (wk_v, WEEK_OFF)):
            iv = idx_ref[sl] + off
            acc0 = acc0 + plsc.load_gather(ps0_v, [iv])
            acc1 = acc1 + plsc.load_gather(ps1_v, [iv])
        rowv = lanes + g * LANES
        plsc.store_scatter(out_v, [rowv, col0], acc0)
        plsc.store_scatter(out_v, [rowv, col1], acc1)
        return carry

    lax.fori_loop(0, GROUPS, group, 0)
    pltpu.sync_copy(out_v, out_hbm.at[pl.ds(base, ROWS_PER_W)])


_sc_call = functools.partial(
    pl.kernel,
    out_type=jax.ShapeDtypeStruct((B, 2), jnp.float32),
    compiler_params=pltpu.CompilerParams(
        needs_layout_passes=False, use_tc_tiling_on_sc=False,
    ),
    mesh=plsc.VectorSubcoreMesh(
        core_axis_name="c", subcore_axis_name="s",
        num_cores=NC, num_subcores=NS,
    ),
    scratch_types=[
        pltpu.VMEM((ROWS_PER_W,), jnp.int32),
        pltpu.VMEM((ROWS_PER_W,), jnp.int32),
        pltpu.VMEM((ROWS_PER_W,), jnp.int32),
        pltpu.VMEM((ROWS_PER_W,), jnp.int32),
        pltpu.VMEM((ROWS_PER_W,), jnp.int32),
        pltpu.VMEM((ROWS_PER_W,), jnp.float32),
        pltpu.VMEM((ROWS_PER_W,), jnp.float32),
        pltpu.VMEM((PS_ROWS,), jnp.float32),
        pltpu.VMEM((PS_ROWS,), jnp.float32),
        pltpu.VMEM((ROWS_PER_W,), jnp.float32),
        pltpu.VMEM((ROWS_PER_W,), jnp.float32),
        pltpu.VMEM((ROWS_PER_W, 2), jnp.float32),
        pltpu.SemaphoreType.DMA,
    ],
)(_sc_body)


def kernel(inputs, call, stand, taxi, hr, week, callemb, standemb, taxiemb,
           hremb, weekemb, W, b):
    f32 = jnp.float32
    i32 = jnp.int32

    z32 = lambda n: jnp.zeros((n, ENUM), f32)
    smcat = jnp.concatenate(
        [standemb, z32(72 - STAND_ROWS), taxiemb, z32(8), hremb, weekemb,
         z32(1)], axis=0)

    d0, d1, ps0, ps1, pc0, pc1 = pl.pallas_call(
        _tc_body,
        grid=(PC_GRID,),
        in_specs=[
            pl.BlockSpec((B_CHUNK, IN_FEATS), lambda i: (i, 0)),
            pl.BlockSpec((2, 240), lambda i: (0, 0)),
            pl.BlockSpec((1, 2), lambda i: (0, 0)),
            pl.BlockSpec((PS_ROWS, ENUM), lambda i: (0, 0)),
            pl.BlockSpec((PC_CHUNK, ENUM), lambda i: (i, 0)),
        ],
        out_specs=[
            pl.BlockSpec((B_CHUNK,), lambda i: (i,)),
            pl.BlockSpec((B_CHUNK,), lambda i: (i,)),
            pl.BlockSpec((PS_ROWS,), lambda i: (0,)),
            pl.BlockSpec((PS_ROWS,), lambda i: (0,)),
            pl.BlockSpec((PC_CHUNK,), lambda i: (i,)),
            pl.BlockSpec((PC_CHUNK,), lambda i: (i,)),
        ],
        out_shape=[
            jax.ShapeDtypeStruct((B,), f32),
            jax.ShapeDtypeStruct((B,), f32),
            jax.ShapeDtypeStruct((PS_ROWS,), f32),
            jax.ShapeDtypeStruct((PS_ROWS,), f32),
            jax.ShapeDtypeStruct((PC_ROWS,), f32),
            jax.ShapeDtypeStruct((PC_ROWS,), f32),
        ],
    )(inputs, W, b.reshape(1, 2), smcat, callemb)

    return _sc_call(d0, d1, call.astype(i32), stand.astype(i32),
                    taxi.astype(i32), hr.astype(i32), week.astype(i32),
                    ps0, ps1, pc0, pc1)


# speedup vs baseline: 3.7069x; 2.2893x over previous
import functools

import jax
import jax.numpy as jnp
from jax import lax
from jax.experimental import pallas as pl
from jax.experimental.pallas import tpu as pltpu
from jax.experimental.pallas import tpu_sc as plsc

B = 16384
ENUM = 32
IN_FEATS = 80

STAND_ROWS = 65
TAXI_OFF = 72
HR_OFF = 1080
WEEK_OFF = 1128
PS_ROWS = 1136

B_CHUNK = 2048
PC_CHUNK = 13312
PC_GRID = 8
PC_ROWS = PC_CHUNK * PC_GRID

NC, NS, LANES = 2, 16, 16
NW = NC * NS
ROWS_PER_W = B // NW
GROUPS = ROWS_PER_W // LANES
IDX_CHUNK = 128
N_CHUNKS = ROWS_PER_W // IDX_CHUNK

_DOT10 = (((1,), (0,)), ((), ()))


def _tc_body(xt_ref, w_ref, b_ref, st_ref, tx_ref, hr_ref, wk_ref, cet_ref,
             d0_ref, d1_ref, ps0_ref, ps1_ref, pc0_ref, pc1_ref):
    f32 = jnp.float32
    w = w_ref[...]

    d = lax.dot_general(w[:, 0:IN_FEATS], xt_ref[...], _DOT10,
                        preferred_element_type=f32)
    d0_ref[...] = d[0] + b_ref[0, 0]
    d1_ref[...] = d[1] + b_ref[0, 1]

    pc = lax.dot_general(w[:, IN_FEATS:IN_FEATS + ENUM], cet_ref[...], _DOT10,
                         preferred_element_type=f32)
    pc0_ref[...] = pc[0]
    pc1_ref[...] = pc[1]

    @pl.when(pl.program_id(0) == 0)
    def _():
        for ref, r0, rows, woff in ((st_ref, 0, STAND_ROWS, 112),
                                    (tx_ref, TAXI_OFF, 1000, 144),
                                    (hr_ref, HR_OFF, 48, 176),
                                    (wk_ref, WEEK_OFF, 7, 208)):
            p = lax.dot_general(w[:, woff:woff + ENUM], ref[...], _DOT10,
                                preferred_element_type=f32)
            ps0_ref[r0:r0 + rows] = p[0]
            ps1_ref[r0:r0 + rows] = p[1]


def _sc_body(d0_hbm, d1_hbm, call_hbm, stand_hbm, taxi_hbm, hr_hbm, week_hbm,
             ps0_hbm, ps1_hbm, pc0_hbm, pc1_hbm, out_hbm,
             call_v, st_v, tx_v, hr_v, wk_v,
             d0_v, d1_v, ps0_v, ps1_v, c0_v, c1_v, out_v, sem):
    wid = lax.axis_index("s") * NC + lax.axis_index("c")
    base = wid * ROWS_PER_W
    bsl = pl.ds(base, ROWS_PER_W)

    stage = [
        pltpu.async_copy(call_hbm.at[bsl], call_v, sem),
        pltpu.async_copy(stand_hbm.at[bsl], st_v, sem),
        pltpu.async_copy(taxi_hbm.at[bsl], tx_v, sem),
        pltpu.async_copy(hr_hbm.at[bsl], hr_v, sem),
        pltpu.async_copy(week_hbm.at[bsl], wk_v, sem),
        pltpu.async_copy(d0_hbm.at[bsl], d0_v, sem),
        pltpu.async_copy(d1_hbm.at[bsl], d1_v, sem),
        pltpu.async_copy(ps0_hbm, ps0_v, sem),
        pltpu.async_copy(ps1_hbm, ps1_v, sem),
    ]
    for c in stage:
        c.wait()

    for k in range(GROUPS):
        sl = pl.ds(k * LANES, LANES)
        call_v[sl] = call_v[sl] + 1

    gathers = []
    for j in range(N_CHUNKS):
        sl = pl.ds(j * IDX_CHUNK, IDX_CHUNK)
        gathers.append(pltpu.async_copy(pc0_hbm.at[call_v.at[sl]], c0_v.at[sl], sem))
        gathers.append(pltpu.async_copy(pc1_hbm.at[call_v.at[sl]], c1_v.at[sl], sem))
    for c in gathers:
        c.wait()

    lanes = lax.iota(jnp.int32, LANES)
    col0 = jnp.zeros((LANES,), jnp.int32)
    col1 = jnp.ones((LANES,), jnp.int32)

    def group(g, carry):
        sl = pl.ds(g * LANES, LANES)
        acc0 = d0_v[sl] + c0_v[sl]
        acc1 = d1_v[sl] + c1_v[sl]
        for idx_ref, off in ((st_v, 1), (tx_v, TAXI_OFF), (hr_v, HR_OFF),
                             (wk_v, WEEK_OFF)):
            iv = idx_ref[sl] + off
            acc0 = acc0 + plsc.load_gather(ps0_v, [iv])
            acc1 = acc1 + plsc.load_gather(ps1_v, [iv])
        rowv = lanes + g * LANES
        plsc.store_scatter(out_v, [rowv, col0], acc0)
        plsc.store_scatter(out_v, [rowv, col1], acc1)
        return carry

    lax.fori_loop(0, GROUPS, group, 0)
    pltpu.sync_copy(out_v, out_hbm.at[bsl])


_sc_call = functools.partial(
    pl.kernel,
    out_type=jax.ShapeDtypeStruct((B, 2), jnp.float32),
    compiler_params=pltpu.CompilerParams(
        needs_layout_passes=False, use_tc_tiling_on_sc=False,
    ),
    mesh=plsc.VectorSubcoreMesh(
        core_axis_name="c", subcore_axis_name="s",
        num_cores=NC, num_subcores=NS,
    ),
    scratch_types=[
        pltpu.VMEM((ROWS_PER_W,), jnp.int32),
        pltpu.VMEM((ROWS_PER_W,), jnp.int32),
        pltpu.VMEM((ROWS_PER_W,), jnp.int32),
        pltpu.VMEM((ROWS_PER_W,), jnp.int32),
        pltpu.VMEM((ROWS_PER_W,), jnp.int32),
        pltpu.VMEM((ROWS_PER_W,), jnp.float32),
        pltpu.VMEM((ROWS_PER_W,), jnp.float32),
        pltpu.VMEM((PS_ROWS,), jnp.float32),
        pltpu.VMEM((PS_ROWS,), jnp.float32),
        pltpu.VMEM((ROWS_PER_W,), jnp.float32),
        pltpu.VMEM((ROWS_PER_W,), jnp.float32),
        pltpu.VMEM((ROWS_PER_W, 2), jnp.float32),
        pltpu.SemaphoreType.DMA,
    ],
)(_sc_body)


def kernel(inputs, call, stand, taxi, hr, week, callemb, standemb, taxiemb,
           hremb, weekemb, W, b):
    f32 = jnp.float32
    i32 = jnp.int32

    d0, d1, ps0, ps1, pc0, pc1 = pl.pallas_call(
        _tc_body,
        grid=(PC_GRID,),
        in_specs=[
            pl.BlockSpec((IN_FEATS, B_CHUNK), lambda i: (0, i)),
            pl.BlockSpec((2, 240), lambda i: (0, 0)),
            pl.BlockSpec((1, 2), lambda i: (0, 0)),
            pl.BlockSpec((ENUM, STAND_ROWS), lambda i: (0, 0)),
            pl.BlockSpec((ENUM, 1000), lambda i: (0, 0)),
            pl.BlockSpec((ENUM, 48), lambda i: (0, 0)),
            pl.BlockSpec((ENUM, 7), lambda i: (0, 0)),
            pl.BlockSpec((ENUM, PC_CHUNK), lambda i: (0, i)),
        ],
        out_specs=[
            pl.BlockSpec((B_CHUNK,), lambda i: (i,)),
            pl.BlockSpec((B_CHUNK,), lambda i: (i,)),
            pl.BlockSpec((PS_ROWS,), lambda i: (0,)),
            pl.BlockSpec((PS_ROWS,), lambda i: (0,)),
            pl.BlockSpec((PC_CHUNK,), lambda i: (i,)),
            pl.BlockSpec((PC_CHUNK,), lambda i: (i,)),
        ],
        out_shape=[
            jax.ShapeDtypeStruct((B,), f32),
            jax.ShapeDtypeStruct((B,), f32),
            jax.ShapeDtypeStruct((PS_ROWS,), f32),
            jax.ShapeDtypeStruct((PS_ROWS,), f32),
            jax.ShapeDtypeStruct((PC_ROWS,), f32),
            jax.ShapeDtypeStruct((PC_ROWS,), f32),
        ],
    )(inputs.T, W, b.reshape(1, 2), standemb.T, taxiemb.T, hremb.T,
      weekemb.T, callemb.T)

    return _sc_call(d0, d1, call.astype(i32), stand.astype(i32),
                    taxi.astype(i32), hr.astype(i32), week.astype(i32),
                    ps0, ps1, pc0, pc1)


# speedup vs baseline: 4.9844x; 1.3446x over previous
import functools

import jax
import jax.numpy as jnp
from jax import lax
from jax.experimental import pallas as pl
from jax.experimental.pallas import tpu as pltpu
from jax.experimental.pallas import tpu_sc as plsc

B = 16384
ENUM = 32
IN_FEATS = 80

STAND_ROWS = 65
TAXI_OFF = 72
HR_OFF = 1080
WEEK_OFF = 1128
PS_ROWS = 1136

B_CHUNK = 2048
PC_CHUNK = 13312
PC_GRID = 8
PC_ROWS = PC_CHUNK * PC_GRID

NC, NS, LANES = 2, 16, 16
NW = NC * NS
ROWS_PER_W = B // NW
GROUPS = ROWS_PER_W // LANES
IDX_CHUNK = 128
N_CHUNKS = ROWS_PER_W // IDX_CHUNK

_DOT10 = (((1,), (0,)), ((), ()))


def _tc_body(xt_ref, w_ref, b_ref, st_ref, tx_ref, hr_ref, wk_ref, cet_ref,
             d0_ref, d1_ref, ps0_ref, ps1_ref, pc0_ref, pc1_ref):
    f32 = jnp.float32
    w = w_ref[...]

    d = lax.dot_general(w[:, 0:IN_FEATS], xt_ref[...], _DOT10,
                        preferred_element_type=f32)
    d0_ref[...] = d[0] + b_ref[0, 0]
    d1_ref[...] = d[1] + b_ref[0, 1]

    pc = lax.dot_general(w[:, IN_FEATS:IN_FEATS + ENUM], cet_ref[...], _DOT10,
                         preferred_element_type=f32)
    pc0_ref[...] = pc[0]
    pc1_ref[...] = pc[1]

    @pl.when(pl.program_id(0) == 0)
    def _():
        for ref, r0, rows, woff in ((st_ref, 0, STAND_ROWS, 112),
                                    (tx_ref, TAXI_OFF, 1000, 144),
                                    (hr_ref, HR_OFF, 48, 176),
                                    (wk_ref, WEEK_OFF, 7, 208)):
            p = lax.dot_general(w[:, woff:woff + ENUM], ref[...], _DOT10,
                                preferred_element_type=f32)
            ps0_ref[r0:r0 + rows] = p[0]
            ps1_ref[r0:r0 + rows] = p[1]


def _sc_body(d0_hbm, d1_hbm, call_hbm, stand_hbm, taxi_hbm, hr_hbm, week_hbm,
             ps0_hbm, ps1_hbm, pc0_hbm, pc1_hbm, out_hbm,
             call_v, st_v, tx_v, hr_v, wk_v,
             d0_v, d1_v, ps0_v, ps1_v, c0_v, c1_v, out0_v, out1_v, sem):
    wid = lax.axis_index("s") * NC + lax.axis_index("c")
    base = wid * ROWS_PER_W
    bsl = pl.ds(base, ROWS_PER_W)

    stage = [
        pltpu.async_copy(call_hbm.at[bsl], call_v, sem),
        pltpu.async_copy(stand_hbm.at[bsl], st_v, sem),
        pltpu.async_copy(taxi_hbm.at[bsl], tx_v, sem),
        pltpu.async_copy(hr_hbm.at[bsl], hr_v, sem),
        pltpu.async_copy(week_hbm.at[bsl], wk_v, sem),
        pltpu.async_copy(d0_hbm.at[bsl], d0_v, sem),
        pltpu.async_copy(d1_hbm.at[bsl], d1_v, sem),
        pltpu.async_copy(ps0_hbm, ps0_v, sem),
        pltpu.async_copy(ps1_hbm, ps1_v, sem),
    ]
    for c in stage:
        c.wait()

    for k in range(GROUPS):
        sl = pl.ds(k * LANES, LANES)
        call_v[sl] = call_v[sl] + 1

    gathers = []
    for j in range(N_CHUNKS):
        sl = pl.ds(j * IDX_CHUNK, IDX_CHUNK)
        gathers.append(pltpu.async_copy(pc0_hbm.at[call_v.at[sl]], c0_v.at[sl], sem))
        gathers.append(pltpu.async_copy(pc1_hbm.at[call_v.at[sl]], c1_v.at[sl], sem))
    for c in gathers:
        c.wait()

    def group(g, carry):
        sl = pl.ds(g * LANES, LANES)
        acc0 = d0_v[sl] + c0_v[sl]
        acc1 = d1_v[sl] + c1_v[sl]
        for idx_ref, off in ((st_v, 1), (tx_v, TAXI_OFF), (hr_v, HR_OFF),
                             (wk_v, WEEK_OFF)):
            iv = idx_ref[sl] + off
            acc0 = acc0 + plsc.load_gather(ps0_v, [iv])
            acc1 = acc1 + plsc.load_gather(ps1_v, [iv])
        out0_v[sl] = acc0
        out1_v[sl] = acc1
        return carry

    lax.fori_loop(0, GROUPS, group, 0)
    pltpu.sync_copy(out0_v, out_hbm.at[0, bsl])
    pltpu.sync_copy(out1_v, out_hbm.at[1, bsl])


_sc_call = functools.partial(
    pl.kernel,
    out_type=jax.ShapeDtypeStruct((2, B), jnp.float32),
    compiler_params=pltpu.CompilerParams(
        needs_layout_passes=False, use_tc_tiling_on_sc=False,
    ),
    mesh=plsc.VectorSubcoreMesh(
        core_axis_name="c", subcore_axis_name="s",
        num_cores=NC, num_subcores=NS,
    ),
    scratch_types=[
        pltpu.VMEM((ROWS_PER_W,), jnp.int32),
        pltpu.VMEM((ROWS_PER_W,), jnp.int32),
        pltpu.VMEM((ROWS_PER_W,), jnp.int32),
        pltpu.VMEM((ROWS_PER_W,), jnp.int32),
        pltpu.VMEM((ROWS_PER_W,), jnp.int32),
        pltpu.VMEM((ROWS_PER_W,), jnp.float32),
        pltpu.VMEM((ROWS_PER_W,), jnp.float32),
        pltpu.VMEM((PS_ROWS,), jnp.float32),
        pltpu.VMEM((PS_ROWS,), jnp.float32),
        pltpu.VMEM((ROWS_PER_W,), jnp.float32),
        pltpu.VMEM((ROWS_PER_W,), jnp.float32),
        pltpu.VMEM((ROWS_PER_W,), jnp.float32),
        pltpu.VMEM((ROWS_PER_W,), jnp.float32),
        pltpu.SemaphoreType.DMA,
    ],
)(_sc_body)


def kernel(inputs, call, stand, taxi, hr, week, callemb, standemb, taxiemb,
           hremb, weekemb, W, b):
    f32 = jnp.float32
    i32 = jnp.int32

    d0, d1, ps0, ps1, pc0, pc1 = pl.pallas_call(
        _tc_body,
        grid=(PC_GRID,),
        in_specs=[
            pl.BlockSpec((IN_FEATS, B_CHUNK), lambda i: (0, i)),
            pl.BlockSpec((2, 240), lambda i: (0, 0)),
            pl.BlockSpec((1, 2), lambda i: (0, 0)),
            pl.BlockSpec((ENUM, STAND_ROWS), lambda i: (0, 0)),
            pl.BlockSpec((ENUM, 1000), lambda i: (0, 0)),
            pl.BlockSpec((ENUM, 48), lambda i: (0, 0)),
            pl.BlockSpec((ENUM, 7), lambda i: (0, 0)),
            pl.BlockSpec((ENUM, PC_CHUNK), lambda i: (0, i)),
        ],
        out_specs=[
            pl.BlockSpec((B_CHUNK,), lambda i: (i,)),
            pl.BlockSpec((B_CHUNK,), lambda i: (i,)),
            pl.BlockSpec((PS_ROWS,), lambda i: (0,)),
            pl.BlockSpec((PS_ROWS,), lambda i: (0,)),
            pl.BlockSpec((PC_CHUNK,), lambda i: (i,)),
            pl.BlockSpec((PC_CHUNK,), lambda i: (i,)),
        ],
        out_shape=[
            jax.ShapeDtypeStruct((B,), f32),
            jax.ShapeDtypeStruct((B,), f32),
            jax.ShapeDtypeStruct((PS_ROWS,), f32),
            jax.ShapeDtypeStruct((PS_ROWS,), f32),
            jax.ShapeDtypeStruct((PC_ROWS,), f32),
            jax.ShapeDtypeStruct((PC_ROWS,), f32),
        ],
    )(inputs.T, W, b.reshape(1, 2), standemb.T, taxiemb.T, hremb.T,
      weekemb.T, callemb.T)

    out_t = _sc_call(d0, d1, call.astype(i32), stand.astype(i32),
                     taxi.astype(i32), hr.astype(i32), week.astype(i32),
                     ps0, ps1, pc0, pc1)
    return out_t.T


# speedup vs baseline: 5.2434x; 1.0520x over previous
import functools

import jax
import jax.numpy as jnp
from jax import lax
from jax.experimental import pallas as pl
from jax.experimental.pallas import tpu as pltpu
from jax.experimental.pallas import tpu_sc as plsc

B = 16384
ENUM = 32
IN_FEATS = 80

STAND_ROWS = 65
TAXI_OFF = 72
HR_OFF = 1080
WEEK_OFF = 1128
PS_ROWS = 1136

B_CHUNK = 2048
PC_CHUNK = 13312
PC_GRID = 8
PC_ROWS = PC_CHUNK * PC_GRID

NC, NS, LANES = 2, 16, 16
NW = NC * NS
ROWS_PER_W = B // NW
GROUPS = ROWS_PER_W // LANES
IDX_CHUNK = 128
N_CHUNKS = ROWS_PER_W // IDX_CHUNK

_DOT10 = (((1,), (0,)), ((), ()))


def _tc_body(xt_ref, w_ref, b_ref, st_ref, tx_ref, hr_ref, wk_ref, cet_ref,
             d0_ref, d1_ref, ps0_ref, ps1_ref, pc0_ref, pc1_ref):
    f32 = jnp.float32
    w = w_ref[...]

    d = lax.dot_general(w[:, 0:IN_FEATS], xt_ref[...], _DOT10,
                        preferred_element_type=f32)
    d0_ref[...] = d[0] + b_ref[0, 0]
    d1_ref[...] = d[1] + b_ref[0, 1]

    pc = lax.dot_general(w[:, IN_FEATS:IN_FEATS + ENUM], cet_ref[...], _DOT10,
                         preferred_element_type=f32)
    pc0_ref[...] = pc[0]
    pc1_ref[...] = pc[1]

    @pl.when(pl.program_id(0) == 0)
    def _():
        for ref, r0, rows, woff in ((st_ref, 0, STAND_ROWS, 112),
                                    (tx_ref, TAXI_OFF, 1000, 144),
                                    (hr_ref, HR_OFF, 48, 176),
                                    (wk_ref, WEEK_OFF, 7, 208)):
            p = lax.dot_general(w[:, woff:woff + ENUM], ref[...], _DOT10,
                                preferred_element_type=f32)
            ps0_ref[r0:r0 + rows] = p[0]
            ps1_ref[r0:r0 + rows] = p[1]


def _sc_body(d0_hbm, d1_hbm, call_hbm, stand_hbm, taxi_hbm, hr_hbm, week_hbm,
             ps0_hbm, ps1_hbm, pc0_hbm, pc1_hbm, out_hbm,
             call_v, st_v, tx_v, hr_v, wk_v,
             d0_v, d1_v, ps0_v, ps1_v, c0_v, c1_v, out0_v, out1_v, sem):
    wid = lax.axis_index("s") * NC + lax.axis_index("c")
    base = wid * ROWS_PER_W
    bsl = pl.ds(base, ROWS_PER_W)

    stage = [
        pltpu.async_copy(call_hbm.at[bsl], call_v, sem),
        pltpu.async_copy(stand_hbm.at[bsl], st_v, sem),
        pltpu.async_copy(taxi_hbm.at[bsl], tx_v, sem),
        pltpu.async_copy(hr_hbm.at[bsl], hr_v, sem),
        pltpu.async_copy(week_hbm.at[bsl], wk_v, sem),
        pltpu.async_copy(d0_hbm.at[bsl], d0_v, sem),
        pltpu.async_copy(d1_hbm.at[bsl], d1_v, sem),
        pltpu.async_copy(ps0_hbm, ps0_v, sem),
        pltpu.async_copy(ps1_hbm, ps1_v, sem),
    ]
    for c in stage:
        c.wait()

    for k in range(GROUPS):
        sl = pl.ds(k * LANES, LANES)
        call_v[sl] = call_v[sl] + 1

    gathers = []
    for j in range(N_CHUNKS):
        sl = pl.ds(j * IDX_CHUNK, IDX_CHUNK)
        gathers.append(pltpu.async_copy(pc0_hbm.at[call_v.at[sl]], c0_v.at[sl], sem))
        gathers.append(pltpu.async_copy(pc1_hbm.at[call_v.at[sl]], c1_v.at[sl], sem))
    for c in gathers:
        c.wait()

    def group(g, carry):
        sl = pl.ds(g * LANES, LANES)
        acc0 = d0_v[sl] + c0_v[sl]
        acc1 = d1_v[sl] + c1_v[sl]
        for idx_ref, off in ((st_v, 1), (tx_v, TAXI_OFF), (hr_v, HR_OFF),
                             (wk_v, WEEK_OFF)):
            iv = idx_ref[sl] + off
            acc0 = acc0 + plsc.load_gather(ps0_v, [iv])
            acc1 = acc1 + plsc.load_gather(ps1_v, [iv])
        out0_v[sl] = acc0
        out1_v[sl] = acc1
        return carry

    lax.fori_loop(0, GROUPS, group, 0)
    outs = []
    for q in range(ROWS_PER_W // 128):
        qsl = pl.ds(q * 128, 128)
        nb = base // 128 + q
        outs.append(pltpu.async_copy(out0_v.at[qsl], out_hbm.at[nb, 0], sem))
        outs.append(pltpu.async_copy(out1_v.at[qsl], out_hbm.at[nb, 1], sem))
    for c in outs:
        c.wait()


_sc_call = functools.partial(
    pl.kernel,
    out_type=jax.ShapeDtypeStruct((B // 128, 2, 128), jnp.float32),
    compiler_params=pltpu.CompilerParams(
        needs_layout_passes=False, use_tc_tiling_on_sc=False,
    ),
    mesh=plsc.VectorSubcoreMesh(
        core_axis_name="c", subcore_axis_name="s",
        num_cores=NC, num_subcores=NS,
    ),
    scratch_types=[
        pltpu.VMEM((ROWS_PER_W,), jnp.int32),
        pltpu.VMEM((ROWS_PER_W,), jnp.int32),
        pltpu.VMEM((ROWS_PER_W,), jnp.int32),
        pltpu.VMEM((ROWS_PER_W,), jnp.int32),
        pltpu.VMEM((ROWS_PER_W,), jnp.int32),
        pltpu.VMEM((ROWS_PER_W,), jnp.float32),
        pltpu.VMEM((ROWS_PER_W,), jnp.float32),
        pltpu.VMEM((PS_ROWS,), jnp.float32),
        pltpu.VMEM((PS_ROWS,), jnp.float32),
        pltpu.VMEM((ROWS_PER_W,), jnp.float32),
        pltpu.VMEM((ROWS_PER_W,), jnp.float32),
        pltpu.VMEM((ROWS_PER_W,), jnp.float32),
        pltpu.VMEM((ROWS_PER_W,), jnp.float32),
        pltpu.SemaphoreType.DMA,
    ],
)(_sc_body)


def kernel(inputs, call, stand, taxi, hr, week, callemb, standemb, taxiemb,
           hremb, weekemb, W, b):
    f32 = jnp.float32
    i32 = jnp.int32

    d0, d1, ps0, ps1, pc0, pc1 = pl.pallas_call(
        _tc_body,
        grid=(PC_GRID,),
        in_specs=[
            pl.BlockSpec((IN_FEATS, B_CHUNK), lambda i: (0, i)),
            pl.BlockSpec((2, 240), lambda i: (0, 0)),
            pl.BlockSpec((1, 2), lambda i: (0, 0)),
            pl.BlockSpec((ENUM, STAND_ROWS), lambda i: (0, 0)),
            pl.BlockSpec((ENUM, 1000), lambda i: (0, 0)),
            pl.BlockSpec((ENUM, 48), lambda i: (0, 0)),
            pl.BlockSpec((ENUM, 7), lambda i: (0, 0)),
            pl.BlockSpec((ENUM, PC_CHUNK), lambda i: (0, i)),
        ],
        out_specs=[
            pl.BlockSpec((B_CHUNK,), lambda i: (i,)),
            pl.BlockSpec((B_CHUNK,), lambda i: (i,)),
            pl.BlockSpec((PS_ROWS,), lambda i: (0,)),
            pl.BlockSpec((PS_ROWS,), lambda i: (0,)),
            pl.BlockSpec((PC_CHUNK,), lambda i: (i,)),
            pl.BlockSpec((PC_CHUNK,), lambda i: (i,)),
        ],
        out_shape=[
            jax.ShapeDtypeStruct((B,), f32),
            jax.ShapeDtypeStruct((B,), f32),
            jax.ShapeDtypeStruct((PS_ROWS,), f32),
            jax.ShapeDtypeStruct((PS_ROWS,), f32),
            jax.ShapeDtypeStruct((PC_ROWS,), f32),
            jax.ShapeDtypeStruct((PC_ROWS,), f32),
        ],
    )(inputs.T, W, b.reshape(1, 2), standemb.T, taxiemb.T, hremb.T,
      weekemb.T, callemb.T)

    out3 = _sc_call(d0, d1, call.astype(i32), stand.astype(i32),
                    taxi.astype(i32), hr.astype(i32), week.astype(i32),
                    ps0, ps1, pc0, pc1)
    return jnp.swapaxes(out3, 1, 2).reshape(B, 2)


# speedup vs baseline: 5.4174x; 1.0332x over previous
import functools

import jax
import jax.numpy as jnp
from jax import lax
from jax.experimental import pallas as pl
from jax.experimental.pallas import tpu as pltpu
from jax.experimental.pallas import tpu_sc as plsc

B = 16384
ENUM = 32
IN_FEATS = 80

STAND_ROWS = 65
TAXI_OFF = 72
HR_OFF = 1080
WEEK_OFF = 1128
PS_ROWS = 1136

B_CHUNK = 2048
PC_CHUNK = 13312
PC_GRID = 8
PC_ROWS = PC_CHUNK * PC_GRID

NC, NS, LANES = 2, 16, 16
NW = NC * NS
ROWS_PER_W = B // NW
GROUPS = ROWS_PER_W // LANES
IDX_CHUNK = 128
N_CHUNKS = ROWS_PER_W // IDX_CHUNK

_DOT10 = (((1,), (0,)), ((), ()))


def _tc_body(xt_ref, w_ref, b_ref, st_ref, tx_ref, hr_ref, wk_ref, cet_ref,
             d0_ref, d1_ref, ps0_ref, ps1_ref, pc0_ref, pc1_ref):
    f32 = jnp.float32
    w = w_ref[...]

    d = lax.dot_general(w[:, 0:IN_FEATS], xt_ref[...], _DOT10,
                        preferred_element_type=f32)
    d0_ref[...] = d[0] + b_ref[0, 0]
    d1_ref[...] = d[1] + b_ref[0, 1]

    pc = lax.dot_general(w[:, IN_FEATS:IN_FEATS + ENUM], cet_ref[...], _DOT10,
                         preferred_element_type=f32)
    pc0_ref[...] = pc[0]
    pc1_ref[...] = pc[1]

    @pl.when(pl.program_id(0) == 0)
    def _():
        for ref, r0, rows, woff in ((st_ref, 0, STAND_ROWS, 112),
                                    (tx_ref, TAXI_OFF, 1000, 144),
                                    (hr_ref, HR_OFF, 48, 176),
                                    (wk_ref, WEEK_OFF, 7, 208)):
            p = lax.dot_general(w[:, woff:woff + ENUM], ref[...], _DOT10,
                                preferred_element_type=f32)
            ps0_ref[r0:r0 + rows] = p[0]
            ps1_ref[r0:r0 + rows] = p[1]


def _sc_body(d0_hbm, d1_hbm, call_hbm, stand_hbm, taxi_hbm, hr_hbm, week_hbm,
             ps0_hbm, ps1_hbm, pc0_hbm, pc1_hbm, out_hbm,
             call_v, st_v, tx_v, hr_v, wk_v,
             d0_v, d1_v, ps0_v, ps1_v, c0_v, c1_v, out0_v, out1_v, sem, sem2):
    wid = lax.axis_index("s") * NC + lax.axis_index("c")
    base = wid * ROWS_PER_W
    bsl = pl.ds(base, ROWS_PER_W)

    call_cp = pltpu.async_copy(call_hbm.at[bsl], call_v, sem)
    stage = [
        pltpu.async_copy(stand_hbm.at[bsl], st_v, sem2),
        pltpu.async_copy(taxi_hbm.at[bsl], tx_v, sem2),
        pltpu.async_copy(hr_hbm.at[bsl], hr_v, sem2),
        pltpu.async_copy(week_hbm.at[bsl], wk_v, sem2),
        pltpu.async_copy(d0_hbm.at[bsl], d0_v, sem2),
        pltpu.async_copy(d1_hbm.at[bsl], d1_v, sem2),
        pltpu.async_copy(ps0_hbm, ps0_v, sem2),
        pltpu.async_copy(ps1_hbm, ps1_v, sem2),
    ]
    call_cp.wait()

    for k in range(GROUPS):
        sl = pl.ds(k * LANES, LANES)
        call_v[sl] = call_v[sl] + 1

    gathers = []
    for j in range(N_CHUNKS):
        sl = pl.ds(j * IDX_CHUNK, IDX_CHUNK)
        gathers.append(pltpu.async_copy(pc0_hbm.at[call_v.at[sl]], c0_v.at[sl], sem))
        gathers.append(pltpu.async_copy(pc1_hbm.at[call_v.at[sl]], c1_v.at[sl], sem))
    for c in stage:
        c.wait()
    for c in gathers:
        c.wait()

    def group(g, carry):
        sl = pl.ds(g * LANES, LANES)
        acc0 = d0_v[sl] + c0_v[sl]
        acc1 = d1_v[sl] + c1_v[sl]
        for idx_ref, off in ((st_v, 1), (tx_v, TAXI_OFF), (hr_v, HR_OFF),
                             (wk_v, WEEK_OFF)):
            iv = idx_ref[sl] + off
            acc0 = acc0 + plsc.load_gather(ps0_v, [iv])
            acc1 = acc1 + plsc.load_gather(ps1_v, [iv])
        out0_v[sl] = acc0
        out1_v[sl] = acc1
        return carry

    lax.fori_loop(0, GROUPS, group, 0)
    outs = []
    for q in range(ROWS_PER_W // 128):
        qsl = pl.ds(q * 128, 128)
        nb = base // 128 + q
        outs.append(pltpu.async_copy(out0_v.at[qsl], out_hbm.at[nb, 0], sem))
        outs.append(pltpu.async_copy(out1_v.at[qsl], out_hbm.at[nb, 1], sem))
    for c in outs:
        c.wait()


_sc_call = functools.partial(
    pl.kernel,
    out_type=jax.ShapeDtypeStruct((B // 128, 2, 128), jnp.float32),
    compiler_params=pltpu.CompilerParams(
        needs_layout_passes=False, use_tc_tiling_on_sc=False,
    ),
    mesh=plsc.VectorSubcoreMesh(
        core_axis_name="c", subcore_axis_name="s",
        num_cores=NC, num_subcores=NS,
    ),
    scratch_types=[
        pltpu.VMEM((ROWS_PER_W,), jnp.int32),
        pltpu.VMEM((ROWS_PER_W,), jnp.int32),
        pltpu.VMEM((ROWS_PER_W,), jnp.int32),
        pltpu.VMEM((ROWS_PER_W,), jnp.int32),
        pltpu.VMEM((ROWS_PER_W,), jnp.int32),
        pltpu.VMEM((ROWS_PER_W,), jnp.float32),
        pltpu.VMEM((ROWS_PER_W,), jnp.float32),
        pltpu.VMEM((PS_ROWS,), jnp.float32),
        pltpu.VMEM((PS_ROWS,), jnp.float32),
        pltpu.VMEM((ROWS_PER_W,), jnp.float32),
        pltpu.VMEM((ROWS_PER_W,), jnp.float32),
        pltpu.VMEM((ROWS_PER_W,), jnp.float32),
        pltpu.VMEM((ROWS_PER_W,), jnp.float32),
        pltpu.SemaphoreType.DMA,
        pltpu.SemaphoreType.DMA,
    ],
)(_sc_body)


def kernel(inputs, call, stand, taxi, hr, week, callemb, standemb, taxiemb,
           hremb, weekemb, W, b):
    f32 = jnp.float32
    i32 = jnp.int32

    d0, d1, ps0, ps1, pc0, pc1 = pl.pallas_call(
        _tc_body,
        grid=(PC_GRID,),
        in_specs=[
            pl.BlockSpec((IN_FEATS, B_CHUNK), lambda i: (0, i)),
            pl.BlockSpec((2, 240), lambda i: (0, 0)),
            pl.BlockSpec((1, 2), lambda i: (0, 0)),
            pl.BlockSpec((ENUM, STAND_ROWS), lambda i: (0, 0)),
            pl.BlockSpec((ENUM, 1000), lambda i: (0, 0)),
            pl.BlockSpec((ENUM, 48), lambda i: (0, 0)),
            pl.BlockSpec((ENUM, 7), lambda i: (0, 0)),
            pl.BlockSpec((ENUM, PC_CHUNK), lambda i: (0, i)),
        ],
        out_specs=[
            pl.BlockSpec((B_CHUNK,), lambda i: (i,)),
            pl.BlockSpec((B_CHUNK,), lambda i: (i,)),
            pl.BlockSpec((PS_ROWS,), lambda i: (0,)),
            pl.BlockSpec((PS_ROWS,), lambda i: (0,)),
            pl.BlockSpec((PC_CHUNK,), lambda i: (i,)),
            pl.BlockSpec((PC_CHUNK,), lambda i: (i,)),
        ],
        out_shape=[
            jax.ShapeDtypeStruct((B,), f32),
            jax.ShapeDtypeStruct((B,), f32),
            jax.ShapeDtypeStruct((PS_ROWS,), f32),
            jax.ShapeDtypeStruct((PS_ROWS,), f32),
            jax.ShapeDtypeStruct((PC_ROWS,), f32),
            jax.ShapeDtypeStruct((PC_ROWS,), f32),
        ],
    )(inputs.T, W, b.reshape(1, 2), standemb.T, taxiemb.T, hremb.T,
      weekemb.T, callemb.T)

    out3 = _sc_call(d0, d1, call.astype(i32), stand.astype(i32),
                    taxi.astype(i32), hr.astype(i32), week.astype(i32),
                    ps0, ps1, pc0, pc1)
    return jnp.swapaxes(out3, 1, 2).reshape(B, 2)
